# Initial kernel scaffold; baseline (speedup 1.0000x reference)
import jax
import jax.numpy as jnp
from jax.experimental import pallas as pl


def kernel(x, table):
    raise NotImplementedError("write your pallas kernel here")


# speedup vs baseline: 4.5725x; 4.5725x over previous
import functools
import math

import jax
import jax.numpy as jnp
from jax import lax
from jax.experimental import pallas as pl
from jax.experimental.pallas import tpu as pltpu
from jax.experimental.pallas import tpu_sc as plsc

_NC = 2
_NS = 16
_NW = _NC * _NS
_LANES = 128
_K = 8


def _make_gather(V, D, B):
    assert B % (_NW * _LANES) == 0
    rows_idx = B // _LANES
    per_w = rows_idx // _NW
    k = _K
    assert per_w % k == 0
    nchunks = per_w // k
    C = k * _LANES
    scale = float(math.sqrt(float(D)))
    mesh = plsc.VectorSubcoreMesh(core_axis_name="c", subcore_axis_name="s")

    @functools.partial(
        pl.kernel,
        mesh=mesh,
        compiler_params=pltpu.CompilerParams(use_tc_tiling_on_sc=False),
        out_type=jax.ShapeDtypeStruct((B, D), jnp.float32),
        scratch_types=[
            pltpu.VMEM((k, _LANES), jnp.int32),
            pltpu.VMEM((C, D), jnp.float32),
            pltpu.SemaphoreType.DMA,
        ],
    )
    def gath(table_hbm, idx_hbm, out_hbm, idx_v, rows_v, sem):
        wid = lax.axis_index("s") * _NC + lax.axis_index("c")

        def chunk(g, carry):
            row0 = wid * per_w + g * k
            base = row0 * _LANES
            pltpu.sync_copy(idx_hbm.at[pl.ds(row0, k)], idx_v)
            cps = [
                pltpu.async_copy(
                    table_hbm.at[idx_v.at[j]],
                    rows_v.at[pl.ds(j * _LANES, _LANES)],
                    sem,
                )
                for j in range(k)
            ]
            for cp in cps:
                cp.wait()

            def srow(i, c2):
                r0 = i * 8
                for u in range(8):
                    for h in range(0, D, 16):
                        v = rows_v[r0 + u, pl.ds(h, 16)]
                        rows_v[r0 + u, pl.ds(h, 16)] = v * scale
                return c2

            lax.fori_loop(0, C // 8, srow, 0)
            pltpu.sync_copy(rows_v, out_hbm.at[pl.ds(base, C)])
            return carry

        lax.fori_loop(0, nchunks, chunk, 0)

    return gath


def kernel(x, table):
    V, D = table.shape
    B = x.size
    xi = x.reshape(-1).astype(jnp.int32).reshape(B // _LANES, _LANES)
    out = _make_gather(V, D, B)(table, xi)
    return out.reshape(*x.shape, D)


# speedup vs baseline: 4.9143x; 1.0747x over previous
import functools
import math

import jax
import jax.numpy as jnp
from jax import lax
from jax.experimental import pallas as pl
from jax.experimental.pallas import tpu as pltpu
from jax.experimental.pallas import tpu_sc as plsc

_NC = 2
_NS = 16
_NW = _NC * _NS
_LANES = 128
_K = 8


def _make_gather(V, D, B):
    assert B % (_NW * _LANES) == 0
    rows_idx = B // _LANES
    per_w = rows_idx // _NW
    k = _K
    assert per_w % (2 * k) == 0
    nchunks = per_w // k
    half = nchunks // 2
    C = k * _LANES
    scale = float(math.sqrt(float(D)))
    mesh = plsc.VectorSubcoreMesh(core_axis_name="c", subcore_axis_name="s")

    @functools.partial(
        pl.kernel,
        mesh=mesh,
        compiler_params=pltpu.CompilerParams(use_tc_tiling_on_sc=False),
        out_type=jax.ShapeDtypeStruct((B, D), jnp.float32),
        scratch_types=[
            pltpu.VMEM((k, _LANES), jnp.int32),
            pltpu.VMEM((k, _LANES), jnp.int32),
            pltpu.VMEM((C, D), jnp.float32),
            pltpu.VMEM((C, D), jnp.float32),
            pltpu.SemaphoreType.DMA,
            pltpu.SemaphoreType.DMA,
            pltpu.SemaphoreType.DMA,
            pltpu.SemaphoreType.DMA,
        ],
    )
    def gath(table_hbm, idx_hbm, out_hbm, idx0, idx1, rows0, rows1,
             g0, g1, s0, s1):
        wid = lax.axis_index("s") * _NC + lax.axis_index("c")
        idxs = (idx0, idx1)
        rowss = (rows0, rows1)
        gsem = (g0, g1)
        ssem = (s0, s1)

        def fire_gather(g, b):
            row0 = wid * per_w + g * k
            pltpu.sync_copy(idx_hbm.at[pl.ds(row0, k)], idxs[b])
            for j in range(k):
                pltpu.async_copy(
                    table_hbm.at[idxs[b].at[j]],
                    rowss[b].at[pl.ds(j * _LANES, _LANES)],
                    gsem[b],
                )

        def drain_gather(b):
            for j in range(k):
                pltpu.make_async_copy(
                    table_hbm.at[idxs[b].at[j]],
                    rowss[b].at[pl.ds(j * _LANES, _LANES)],
                    gsem[b],
                ).wait()

        def fire_store(g, b):
            base = (wid * per_w + g * k) * _LANES
            pltpu.async_copy(rowss[b], out_hbm.at[pl.ds(base, C)], ssem[b])

        def drain_store(b):
            pltpu.make_async_copy(rowss[b], out_hbm.at[pl.ds(0, C)],
                                  ssem[b]).wait()

        def scale_buf(b):
            r = rowss[b]

            def srow(i, c2):
                r0 = i * 16
                for u in range(16):
                    for h in range(0, D, 16):
                        v = r[r0 + u, pl.ds(h, 16)]
                        r[r0 + u, pl.ds(h, 16)] = v * scale
                return c2

            lax.fori_loop(0, C // 16, srow, 0)

        fire_gather(0, 0)

        def body(i, carry):
            c0 = 2 * i
            c1 = c0 + 1

            @pl.when(i > 0)
            def _():
                drain_store(1)

            fire_gather(c1, 1)
            drain_gather(0)
            scale_buf(0)
            fire_store(c0, 0)

            @pl.when(i + 1 < half)
            def _():
                drain_store(0)
                fire_gather(c0 + 2, 0)

            drain_gather(1)
            scale_buf(1)
            fire_store(c1, 1)
            return carry

        lax.fori_loop(0, half, body, 0)
        drain_store(0)
        drain_store(1)

    return gath


def kernel(x, table):
    V, D = table.shape
    B = x.size
    xi = x.reshape(-1).astype(jnp.int32).reshape(B // _LANES, _LANES)
    out = _make_gather(V, D, B)(table, xi)
    return out.reshape(*x.shape, D)
